# baseline (speedup 1.0000x reference)
import functools

import jax
import jax.numpy as jnp
from jax import lax
from jax.experimental import pallas as pl
from jax.experimental.pallas import tpu as pltpu
from jax.experimental.pallas import tpu_sc as plsc

N = 10000
D = 128
H = 128
C = 5
VH = 32
E = 320000

NC = 2
NS = 16
NW = NC * NS
BLK = 128
NBLK = 80
EPW = NBLK * BLK
EPAD = EPW * NW
NPAD = 10240
RPT = NPAD // NS
ZR = 64
DW = 16
CH = 4
NCHUNK = NBLK // CH

_mesh = plsc.VectorSubcoreMesh(core_axis_name="c", subcore_axis_name="s",
                               num_cores=NC, num_subcores=NS)


def _fill(ref, rows, width, value):
  vec = jnp.full((16,), value, jnp.float32)
  ncol = width // 16

  def body(i, _):
    r = i // ncol
    cc = i % ncol
    ref[r, pl.ds(cc * 16, 16)] = vec
    return 0

  lax.fori_loop(0, rows * ncol, body, 0)


def _make_sc_deg():
  @functools.partial(
      pl.kernel,
      out_type=jax.ShapeDtypeStruct((3, NC, NPAD, DW), jnp.float32),
      mesh=_mesh,
      scratch_types=[
          pltpu.VMEM((NBLK, BLK), jnp.int32),
          pltpu.VMEM((BLK, DW), jnp.float32),
          pltpu.VMEM((ZR, DW), jnp.float32),
          pltpu.VMEM_SHARED((NPAD, DW), jnp.float32),
          pltpu.SemaphoreType.DMA,
          pltpu.SemaphoreType.DMA,
      ],
  )
  def k(dsts, out, didx, ones, zbuf, acc, sems, semz):
    c = lax.axis_index("c")
    s = lax.axis_index("s")
    wid = s * NC + c
    rbase = s * RPT
    _fill(ones, BLK, DW, 1.0)
    _fill(zbuf, ZR, DW, 0.0)
    nz = RPT // ZR
    INFLIGHT = 8

    def wait_one():
      pltpu.make_async_copy(ones, acc.at[didx.at[0]], sems).wait()

    for o in range(3):
      for t in range(nz):
        pltpu.async_copy(zbuf, acc.at[pl.ds(rbase + t * ZR, ZR)], semz)
      for t in range(nz):
        pltpu.make_async_copy(zbuf, acc.at[pl.ds(rbase, ZR)], semz).wait()
      plsc.subcore_barrier()
      pltpu.sync_copy(dsts.at[o, wid], didx)

      def body(j, _):
        @pl.when(j >= INFLIGHT)
        def _():
          wait_one()
        pltpu.async_copy(ones, acc.at[didx.at[j]], sems, add=True)
        return 0

      lax.fori_loop(0, NBLK, body, 0)
      for _ in range(INFLIGHT):
        wait_one()
      plsc.subcore_barrier()
      pltpu.sync_copy(acc.at[pl.ds(rbase, RPT)],
                      out.at[o, c, pl.ds(rbase, RPT)])
      plsc.subcore_barrier()

  return k


def _make_sc_scatter(DD, NBUF, ZRR, o):
  @functools.partial(
      pl.kernel,
      out_type=jax.ShapeDtypeStruct((NC, NPAD, DD), jnp.float32),
      mesh=_mesh,
      compiler_params=pltpu.CompilerParams(use_tc_tiling_on_sc=(DD == D)),
      scratch_types=[
          pltpu.VMEM((CH, BLK), jnp.int32),
          pltpu.VMEM((NBLK, BLK), jnp.int32),
          [pltpu.VMEM((BLK, DD), jnp.float32) for _ in range(NBUF)],
          pltpu.VMEM((ZRR, DD), jnp.float32),
          pltpu.VMEM_SHARED((NPAD, DD), jnp.float32),
          [pltpu.SemaphoreType.DMA for _ in range(NBUF)],
          [pltpu.SemaphoreType.DMA for _ in range(NBUF)],
          pltpu.SemaphoreType.DMA,
      ],
  )
  def k(gt, srcs, dsts, out, sidx, didx, rows, zbuf, acc, semg, sems, semz):
    c = lax.axis_index("c")
    s = lax.axis_index("s")
    wid = s * NC + c
    rbase = s * RPT
    _fill(zbuf, ZRR, DD, 0.0)
    nz = RPT // ZRR

    def wait_g(p):
      pltpu.make_async_copy(gt.at[sidx.at[0]], rows[p], semg[p]).wait()

    def wait_s(p):
      pltpu.make_async_copy(rows[p], acc.at[didx.at[0]], sems[p]).wait()

    for t in range(nz):
      pltpu.async_copy(zbuf, acc.at[pl.ds(rbase + t * ZRR, ZRR)], semz)
    for t in range(nz):
      pltpu.make_async_copy(zbuf, acc.at[pl.ds(rbase, ZRR)], semz).wait()
    plsc.subcore_barrier()

    pltpu.sync_copy(dsts.at[o, wid], didx)
    pltpu.sync_copy(srcs.at[o, wid, pl.ds(0, CH)], sidx)
    pltpu.async_copy(gt.at[sidx.at[0]], rows[0], semg[0])

    def body(ci, _):
      for j in range(CH):
        p = j % NBUF
        q = (j + 1) % NBUF
        b = ci * CH + j
        wait_g(p)
        pltpu.async_copy(rows[p], acc.at[didx.at[b]], sems[p], add=True)
        if j < CH - 1:
          @pl.when(b + 1 >= NBUF)
          def _():
            wait_s(q)
          pltpu.async_copy(gt.at[sidx.at[j + 1]], rows[q], semg[q])
        else:
          @pl.when(ci < NCHUNK - 1)
          def _():
            pltpu.sync_copy(srcs.at[o, wid, pl.ds((ci + 1) * CH, CH)], sidx)
            wait_s(q)
            pltpu.async_copy(gt.at[sidx.at[0]], rows[q], semg[q])
      return 0

    lax.fori_loop(0, NCHUNK, body, 0)
    for p in range(NBUF):
      wait_s(p)
    plsc.subcore_barrier()
    pltpu.sync_copy(acc.at[pl.ds(rbase, RPT)], out.at[c, pl.ds(rbase, RPT)])

  return k


_sc_deg_k = _make_sc_deg()
_sc_l1_ks = [_make_sc_scatter(D, 2, 32, o) for o in range(3)]
_sc_l2_ks = [_make_sc_scatter(DW, 4, ZR, o) for o in range(3)]

R = 2048
NRB = NPAD // R


def _tc_mm_body(x_ref, w1_ref, h1_ref):
  h1_ref[0] = jnp.dot(x_ref[0], w1_ref[0], preferred_element_type=jnp.float32)


def _tc_mm(xs, W1s):
  return pl.pallas_call(
      _tc_mm_body,
      grid=(3, NRB),
      in_specs=[
          pl.BlockSpec((1, R, D), lambda o, i: (o, i, 0)),
          pl.BlockSpec((1, D, H), lambda o, i: (o, 0, 0)),
      ],
      out_specs=pl.BlockSpec((1, R, H), lambda o, i: (o, i, 0)),
      out_shape=jax.ShapeDtypeStruct((3, NPAD, H), jnp.float32),
  )(xs, W1s)


def _tc1_body(h1_ref, deg_ref, g1_ref, dinv_ref):
  deg = deg_ref[0, :, 0] + deg_ref[1, :, 0] + 1.0
  dv = lax.rsqrt(deg)
  g1_ref[...] = dv[:, None] * h1_ref[0]
  dinv_ref[:, 0] = dv


def _tc1(o, h1s, degacc):
  return pl.pallas_call(
      _tc1_body,
      grid=(NRB,),
      in_specs=[
          pl.BlockSpec((1, R, H), lambda i, o=o: (o, i, 0)),
          pl.BlockSpec((NC, R, DW), lambda i: (0, i, 0)),
      ],
      out_specs=[
          pl.BlockSpec((R, H), lambda i: (i, 0)),
          pl.BlockSpec((R, 1), lambda i: (i, 0)),
      ],
      out_shape=[
          jax.ShapeDtypeStruct((NPAD, H), jnp.float32),
          jax.ShapeDtypeStruct((NPAD, 1), jnp.float32),
      ],
  )(h1s, degacc)


def _tc2_body(acc_ref, g1_ref, dinv_ref, b1_ref, w2_ref, g2_ref):
  dv = dinv_ref[:, 0]
  out1 = dv[:, None] * (acc_ref[0] + acc_ref[1] + g1_ref[...])
  out1 = out1 + b1_ref[0, 0][None, :]
  h = jnp.maximum(out1, 0.0)
  t = jnp.dot(h, w2_ref[0], preferred_element_type=jnp.float32)
  g2_ref[...] = dv[:, None] * t


def _tc2(o, acc1, g1, dinv, b1s, W2p):
  return pl.pallas_call(
      _tc2_body,
      grid=(NRB,),
      in_specs=[
          pl.BlockSpec((NC, R, H), lambda i: (0, i, 0)),
          pl.BlockSpec((R, H), lambda i: (i, 0)),
          pl.BlockSpec((R, 1), lambda i: (i, 0)),
          pl.BlockSpec((1, 1, H), lambda i, o=o: (o, 0, 0)),
          pl.BlockSpec((1, H, DW), lambda i, o=o: (o, 0, 0)),
      ],
      out_specs=pl.BlockSpec((R, DW), lambda i: (i, 0)),
      out_shape=jax.ShapeDtypeStruct((NPAD, DW), jnp.float32),
  )(acc1, g1, dinv, b1s, W2p)


def _tc3_body(a0_ref, a1_ref, a2_ref, g0_ref, g1_ref, g2_ref,
              d0_ref, d1_ref, d2_ref, b2_ref, wv1_ref, bv1_ref,
              wv2_ref, bv2_ref, out_ref):
  accs = (a0_ref, a1_ref, a2_ref)
  gs = (g0_ref, g1_ref, g2_ref)
  dvs = (d0_ref, d1_ref, d2_ref)
  ps = []
  for o in range(3):
    v = accs[o][0] + accs[o][1] + gs[o][...]
    v = dvs[o][:, 0][:, None] * v + b2_ref[o][None, :]
    ps.append(jax.nn.sigmoid(v[:, :C]))
  m = lax.broadcasted_iota(jnp.int32, (C, 128), 1)
  r = lax.broadcasted_iota(jnp.int32, (C, 128), 0)
  s0 = (r == m // 25).astype(jnp.float32)
  s1 = (r == (m // 5) % 5).astype(jnp.float32)
  s2 = (r == m % 5).astype(jnp.float32)
  a = jnp.dot(ps[0], s0, preferred_element_type=jnp.float32)
  b = jnp.dot(ps[1], s1, preferred_element_type=jnp.float32)
  cc = jnp.dot(ps[2], s2, preferred_element_type=jnp.float32)
  t = a * b * cc
  q = jnp.dot(t, wv1_ref[...], preferred_element_type=jnp.float32)
  q = q + bv1_ref[0][None, :]
  z = jnp.where(q >= 0.0, q, 0.25 * q)
  out_ref[...] = jnp.dot(z, wv2_ref[...],
                         preferred_element_type=jnp.float32) + bv2_ref[0][None, :]


def _tc3(acc2s, g2s, dinvs, b2p, Wv1p, bv1r, Wv2, bv2r):
  return pl.pallas_call(
      _tc3_body,
      grid=(NRB,),
      in_specs=(
          [pl.BlockSpec((NC, R, DW), lambda i: (0, i, 0)) for _ in range(3)]
          + [pl.BlockSpec((R, DW), lambda i: (i, 0)) for _ in range(3)]
          + [pl.BlockSpec((R, 1), lambda i: (i, 0)) for _ in range(3)]
          + [
              pl.BlockSpec((3, DW), lambda i: (0, 0)),
              pl.BlockSpec((128, VH), lambda i: (0, 0)),
              pl.BlockSpec((1, VH), lambda i: (0, 0)),
              pl.BlockSpec((VH, C), lambda i: (0, 0)),
              pl.BlockSpec((1, C), lambda i: (0, 0)),
          ]
      ),
      out_specs=pl.BlockSpec((R, C), lambda i: (i, 0)),
      out_shape=jax.ShapeDtypeStruct((NPAD, C), jnp.float32),
  )(*acc2s, *g2s, *dinvs, b2p, Wv1p, bv1r, Wv2, bv2r)


def kernel(x_mrna, x_meth, x_mirna,
           W1_mrna, b1_mrna, W2_mrna, b2_mrna,
           W1_meth, b1_meth, W2_meth, b2_meth,
           W1_mirna, b1_mirna, W2_mirna, b2_mirna,
           Wv1, bv1, Wv2, bv2,
           ei_mrna, ei_meth, ei_mirna):
  xs = jnp.concatenate(
      [jnp.stack([x_mrna, x_meth, x_mirna]),
       jnp.zeros((3, NPAD - N, D), jnp.float32)], axis=1)
  W1s = jnp.stack([W1_mrna, W1_meth, W1_mirna])
  b1s = jnp.stack([b1_mrna, b1_meth, b1_mirna])[:, None, :]
  W2p = jnp.zeros((3, H, DW), jnp.float32).at[:, :, :C].set(
      jnp.stack([W2_mrna, W2_meth, W2_mirna]))
  b2p = jnp.zeros((3, DW), jnp.float32).at[:, :C].set(
      jnp.stack([b2_mrna, b2_meth, b2_mirna]))
  Wv1p = jnp.zeros((128, VH), jnp.float32).at[:C ** 3, :].set(Wv1)

  npad = EPAD - E
  pad_src = (jnp.arange(npad, dtype=jnp.int32) % N)
  pad_dst = N + (jnp.arange(npad, dtype=jnp.int32) % (NPAD - N))
  srcs_r = jnp.stack([
      jnp.concatenate([ei[0], pad_src]) for ei in (ei_mrna, ei_meth, ei_mirna)
  ]).reshape(3, NW, NBLK, BLK)
  dsts_r = jnp.stack([
      jnp.concatenate([ei[1], pad_dst]) for ei in (ei_mrna, ei_meth, ei_mirna)
  ]).reshape(3, NW, NBLK, BLK)

  degaccs = _sc_deg_k(dsts_r)
  h1s = _tc_mm(xs, W1s)
  acc2s, g2s, dinvs = [], [], []
  for o in range(3):
    g1, dinv = _tc1(o, h1s, degaccs[o])
    acc1 = _sc_l1_ks[o](g1, srcs_r, dsts_r)
    g2 = _tc2(o, acc1, g1, dinv, b1s, W2p)
    acc2 = _sc_l2_ks[o](g2, srcs_r, dsts_r)
    acc2s.append(acc2)
    g2s.append(g2)
    dinvs.append(dinv)
  out = _tc3(acc2s, g2s, dinvs, b2p, Wv1p, bv1[None, :],
             Wv2, bv2[None, :])
  return out[:N]


# speedup vs baseline: 1.0247x; 1.0247x over previous
import functools

import jax
import jax.numpy as jnp
from jax import lax
from jax.experimental import pallas as pl
from jax.experimental.pallas import tpu as pltpu
from jax.experimental.pallas import tpu_sc as plsc

N = 10000
D = 128
H = 128
C = 5
VH = 32
E = 320000

NC = 2
NS = 16
NW = NC * NS
BLK = 128
NBLK = 80
EPW = NBLK * BLK
EPAD = EPW * NW
NPAD = 10240
RPT = NPAD // NS
ZR = 64
DW = 16
CH = 4
NCHUNK = NBLK // CH

_mesh = plsc.VectorSubcoreMesh(core_axis_name="c", subcore_axis_name="s",
                               num_cores=NC, num_subcores=NS)


def _fill(ref, rows, width, value):
  vec = jnp.full((16,), value, jnp.float32)
  ncol = width // 16

  def body(i, _):
    r = i // ncol
    cc = i % ncol
    ref[r, pl.ds(cc * 16, 16)] = vec
    return 0

  lax.fori_loop(0, rows * ncol, body, 0)


def _make_sc_deg(o):
  @functools.partial(
      pl.kernel,
      out_type=jax.ShapeDtypeStruct((NC, NPAD, DW), jnp.float32),
      mesh=_mesh,
      scratch_types=[
          pltpu.VMEM((NBLK, BLK), jnp.int32),
          pltpu.VMEM((BLK, DW), jnp.float32),
          pltpu.VMEM((ZR, DW), jnp.float32),
          pltpu.VMEM_SHARED((NPAD, DW), jnp.float32),
          pltpu.SemaphoreType.DMA,
          pltpu.SemaphoreType.DMA,
      ],
  )
  def k(dsts, out, didx, ones, zbuf, acc, sems, semz):
    c = lax.axis_index("c")
    s = lax.axis_index("s")
    wid = s * NC + c
    rbase = s * RPT
    _fill(ones, BLK, DW, 1.0)
    _fill(zbuf, ZR, DW, 0.0)
    nz = RPT // ZR
    INFLIGHT = 8

    def wait_one():
      pltpu.make_async_copy(ones, acc.at[didx.at[0]], sems).wait()

    for t in range(nz):
      pltpu.async_copy(zbuf, acc.at[pl.ds(rbase + t * ZR, ZR)], semz)
    for t in range(nz):
      pltpu.make_async_copy(zbuf, acc.at[pl.ds(rbase, ZR)], semz).wait()
    plsc.subcore_barrier()
    pltpu.sync_copy(dsts.at[o, wid], didx)

    def body(j, _):
      @pl.when(j >= INFLIGHT)
      def _():
        wait_one()
      pltpu.async_copy(ones, acc.at[didx.at[j]], sems, add=True)
      return 0

    lax.fori_loop(0, NBLK, body, 0)
    for _ in range(INFLIGHT):
      wait_one()
    plsc.subcore_barrier()
    pltpu.sync_copy(acc.at[pl.ds(rbase, RPT)], out.at[c, pl.ds(rbase, RPT)])

  return k


def _make_sc_scatter(DD, NBUF, ZRR, o):
  @functools.partial(
      pl.kernel,
      out_type=jax.ShapeDtypeStruct((NC, NPAD, DD), jnp.float32),
      mesh=_mesh,
      compiler_params=pltpu.CompilerParams(use_tc_tiling_on_sc=(DD == D)),
      scratch_types=[
          pltpu.VMEM((CH, BLK), jnp.int32),
          pltpu.VMEM((NBLK, BLK), jnp.int32),
          [pltpu.VMEM((BLK, DD), jnp.float32) for _ in range(NBUF)],
          pltpu.VMEM((ZRR, DD), jnp.float32),
          pltpu.VMEM_SHARED((NPAD, DD), jnp.float32),
          [pltpu.SemaphoreType.DMA for _ in range(NBUF)],
          [pltpu.SemaphoreType.DMA for _ in range(NBUF)],
          pltpu.SemaphoreType.DMA,
      ],
  )
  def k(gt, srcs, dsts, out, sidx, didx, rows, zbuf, acc, semg, sems, semz):
    c = lax.axis_index("c")
    s = lax.axis_index("s")
    wid = s * NC + c
    rbase = s * RPT
    _fill(zbuf, ZRR, DD, 0.0)
    nz = RPT // ZRR

    def wait_g(p):
      pltpu.make_async_copy(gt.at[sidx.at[0]], rows[p], semg[p]).wait()

    def wait_s(p):
      pltpu.make_async_copy(rows[p], acc.at[didx.at[0]], sems[p]).wait()

    for t in range(nz):
      pltpu.async_copy(zbuf, acc.at[pl.ds(rbase + t * ZRR, ZRR)], semz)
    for t in range(nz):
      pltpu.make_async_copy(zbuf, acc.at[pl.ds(rbase, ZRR)], semz).wait()
    plsc.subcore_barrier()

    pltpu.sync_copy(dsts.at[o, wid], didx)
    pltpu.sync_copy(srcs.at[o, wid, pl.ds(0, CH)], sidx)
    pltpu.async_copy(gt.at[sidx.at[0]], rows[0], semg[0])

    def body(ci, _):
      for j in range(CH):
        p = j % NBUF
        q = (j + 1) % NBUF
        b = ci * CH + j
        wait_g(p)
        pltpu.async_copy(rows[p], acc.at[didx.at[b]], sems[p], add=True)
        if j < CH - 1:
          @pl.when(b + 1 >= NBUF)
          def _():
            wait_s(q)
          pltpu.async_copy(gt.at[sidx.at[j + 1]], rows[q], semg[q])
        else:
          @pl.when(ci < NCHUNK - 1)
          def _():
            pltpu.sync_copy(srcs.at[o, wid, pl.ds((ci + 1) * CH, CH)], sidx)
            wait_s(q)
            pltpu.async_copy(gt.at[sidx.at[0]], rows[q], semg[q])
      return 0

    lax.fori_loop(0, NCHUNK, body, 0)
    for p in range(NBUF):
      wait_s(p)
    plsc.subcore_barrier()
    pltpu.sync_copy(acc.at[pl.ds(rbase, RPT)], out.at[c, pl.ds(rbase, RPT)])

  return k


_sc_deg_ks = [_make_sc_deg(o) for o in range(3)]
_sc_l1_ks = [_make_sc_scatter(D, 2, 32, o) for o in range(3)]
_sc_l2_ks = [_make_sc_scatter(DW, 4, ZR, o) for o in range(3)]

R = 2048
NRB = NPAD // R


def _tc_mm_body(x_ref, w1_ref, h1_ref):
  h1_ref[0] = jnp.dot(x_ref[0], w1_ref[0], preferred_element_type=jnp.float32)


def _tc_mm(xs, W1s):
  return pl.pallas_call(
      _tc_mm_body,
      grid=(3, NRB),
      in_specs=[
          pl.BlockSpec((1, R, D), lambda o, i: (o, i, 0)),
          pl.BlockSpec((1, D, H), lambda o, i: (o, 0, 0)),
      ],
      out_specs=pl.BlockSpec((1, R, H), lambda o, i: (o, i, 0)),
      out_shape=jax.ShapeDtypeStruct((3, NPAD, H), jnp.float32),
  )(xs, W1s)


def _tc1_body(h1_ref, deg_ref, g1_ref, dinv_ref):
  deg = deg_ref[0, :, 0] + deg_ref[1, :, 0] + 1.0
  dv = lax.rsqrt(deg)
  g1_ref[...] = dv[:, None] * h1_ref[0]
  dinv_ref[:, 0] = dv


def _tc1(o, h1s, degacc):
  return pl.pallas_call(
      _tc1_body,
      grid=(NRB,),
      in_specs=[
          pl.BlockSpec((1, R, H), lambda i, o=o: (o, i, 0)),
          pl.BlockSpec((NC, R, DW), lambda i: (0, i, 0)),
      ],
      out_specs=[
          pl.BlockSpec((R, H), lambda i: (i, 0)),
          pl.BlockSpec((R, 1), lambda i: (i, 0)),
      ],
      out_shape=[
          jax.ShapeDtypeStruct((NPAD, H), jnp.float32),
          jax.ShapeDtypeStruct((NPAD, 1), jnp.float32),
      ],
  )(h1s, degacc)


def _tc2_body(acc_ref, g1_ref, dinv_ref, b1_ref, w2_ref, g2_ref):
  dv = dinv_ref[:, 0]
  out1 = dv[:, None] * (acc_ref[0] + acc_ref[1] + g1_ref[...])
  out1 = out1 + b1_ref[0, 0][None, :]
  h = jnp.maximum(out1, 0.0)
  t = jnp.dot(h, w2_ref[0], preferred_element_type=jnp.float32)
  g2_ref[...] = dv[:, None] * t


def _tc2(o, acc1, g1, dinv, b1s, W2p):
  return pl.pallas_call(
      _tc2_body,
      grid=(NRB,),
      in_specs=[
          pl.BlockSpec((NC, R, H), lambda i: (0, i, 0)),
          pl.BlockSpec((R, H), lambda i: (i, 0)),
          pl.BlockSpec((R, 1), lambda i: (i, 0)),
          pl.BlockSpec((1, 1, H), lambda i, o=o: (o, 0, 0)),
          pl.BlockSpec((1, H, DW), lambda i, o=o: (o, 0, 0)),
      ],
      out_specs=pl.BlockSpec((R, DW), lambda i: (i, 0)),
      out_shape=jax.ShapeDtypeStruct((NPAD, DW), jnp.float32),
  )(acc1, g1, dinv, b1s, W2p)


def _tc3_body(a0_ref, a1_ref, a2_ref, g0_ref, g1_ref, g2_ref,
              d0_ref, d1_ref, d2_ref, b2_ref, wv1_ref, bv1_ref,
              wv2_ref, bv2_ref, out_ref):
  accs = (a0_ref, a1_ref, a2_ref)
  gs = (g0_ref, g1_ref, g2_ref)
  dvs = (d0_ref, d1_ref, d2_ref)
  ps = []
  for o in range(3):
    v = accs[o][0] + accs[o][1] + gs[o][...]
    v = dvs[o][:, 0][:, None] * v + b2_ref[o][None, :]
    ps.append(jax.nn.sigmoid(v[:, :C]))
  m = lax.broadcasted_iota(jnp.int32, (C, 128), 1)
  r = lax.broadcasted_iota(jnp.int32, (C, 128), 0)
  s0 = (r == m // 25).astype(jnp.float32)
  s1 = (r == (m // 5) % 5).astype(jnp.float32)
  s2 = (r == m % 5).astype(jnp.float32)
  a = jnp.dot(ps[0], s0, preferred_element_type=jnp.float32)
  b = jnp.dot(ps[1], s1, preferred_element_type=jnp.float32)
  cc = jnp.dot(ps[2], s2, preferred_element_type=jnp.float32)
  t = a * b * cc
  q = jnp.dot(t, wv1_ref[...], preferred_element_type=jnp.float32)
  q = q + bv1_ref[0][None, :]
  z = jnp.where(q >= 0.0, q, 0.25 * q)
  out_ref[...] = jnp.dot(z, wv2_ref[...],
                         preferred_element_type=jnp.float32) + bv2_ref[0][None, :]


def _tc3(acc2s, g2s, dinvs, b2p, Wv1p, bv1r, Wv2, bv2r):
  return pl.pallas_call(
      _tc3_body,
      grid=(NRB,),
      in_specs=(
          [pl.BlockSpec((NC, R, DW), lambda i: (0, i, 0)) for _ in range(3)]
          + [pl.BlockSpec((R, DW), lambda i: (i, 0)) for _ in range(3)]
          + [pl.BlockSpec((R, 1), lambda i: (i, 0)) for _ in range(3)]
          + [
              pl.BlockSpec((3, DW), lambda i: (0, 0)),
              pl.BlockSpec((128, VH), lambda i: (0, 0)),
              pl.BlockSpec((1, VH), lambda i: (0, 0)),
              pl.BlockSpec((VH, C), lambda i: (0, 0)),
              pl.BlockSpec((1, C), lambda i: (0, 0)),
          ]
      ),
      out_specs=pl.BlockSpec((R, C), lambda i: (i, 0)),
      out_shape=jax.ShapeDtypeStruct((NPAD, C), jnp.float32),
  )(*acc2s, *g2s, *dinvs, b2p, Wv1p, bv1r, Wv2, bv2r)


def kernel(x_mrna, x_meth, x_mirna,
           W1_mrna, b1_mrna, W2_mrna, b2_mrna,
           W1_meth, b1_meth, W2_meth, b2_meth,
           W1_mirna, b1_mirna, W2_mirna, b2_mirna,
           Wv1, bv1, Wv2, bv2,
           ei_mrna, ei_meth, ei_mirna):
  xs = jnp.concatenate(
      [jnp.stack([x_mrna, x_meth, x_mirna]),
       jnp.zeros((3, NPAD - N, D), jnp.float32)], axis=1)
  W1s = jnp.stack([W1_mrna, W1_meth, W1_mirna])
  b1s = jnp.stack([b1_mrna, b1_meth, b1_mirna])[:, None, :]
  W2p = jnp.zeros((3, H, DW), jnp.float32).at[:, :, :C].set(
      jnp.stack([W2_mrna, W2_meth, W2_mirna]))
  b2p = jnp.zeros((3, DW), jnp.float32).at[:, :C].set(
      jnp.stack([b2_mrna, b2_meth, b2_mirna]))
  Wv1p = jnp.zeros((128, VH), jnp.float32).at[:C ** 3, :].set(Wv1)

  npad = EPAD - E
  pad_src = (jnp.arange(npad, dtype=jnp.int32) % N)
  pad_dst = N + (jnp.arange(npad, dtype=jnp.int32) % (NPAD - N))
  srcs_r = jnp.stack([
      jnp.concatenate([ei[0], pad_src]) for ei in (ei_mrna, ei_meth, ei_mirna)
  ]).reshape(3, NW, NBLK, BLK)
  dsts_r = jnp.stack([
      jnp.concatenate([ei[1], pad_dst]) for ei in (ei_mrna, ei_meth, ei_mirna)
  ]).reshape(3, NW, NBLK, BLK)

  h1s = _tc_mm(xs, W1s)
  acc2s, g2s, dinvs = [], [], []
  for o in range(3):
    degacc = _sc_deg_ks[o](dsts_r)
    g1, dinv = _tc1(o, h1s, degacc)
    acc1 = _sc_l1_ks[o](g1, srcs_r, dsts_r)
    g2 = _tc2(o, acc1, g1, dinv, b1s, W2p)
    acc2 = _sc_l2_ks[o](g2, srcs_r, dsts_r)
    acc2s.append(acc2)
    g2s.append(g2)
    dinvs.append(dinv)
  out = _tc3(acc2s, g2s, dinvs, b2p, Wv1p, bv1[None, :],
             Wv2, bv2[None, :])
  return out[:N]
